# baseline (speedup 1.0000x reference)
import jax
import jax.numpy as jnp
from jax import lax
from jax.experimental import pallas as pl
from jax.experimental.pallas import tpu as pltpu
from jax.experimental.pallas import tpu_sc as plsc

N = 100000
B = 4096
K = 16
D = 128
DE = 16
DT = 100
H = 2
DH = D // H
E = B + B * K
NT = 5

_NC = 2
_NS = 16
_NW = _NC * _NS
_CH = 64
_PER_W = E // _NW
_NCHUNK = _PER_W // _CH


def _sc_gather_body(nodes, mem, mbox, nfeat, aux, o_big,
                    idx_all, b0, b1, gsem0, gsem1, osem0, osem1):
    cid = lax.axis_index("c")
    sid = lax.axis_index("s")
    wid = sid * _NC + cid
    base = wid * _PER_W
    pltpu.sync_copy(nodes.at[pl.ds(base, _PER_W)], idx_all)

    srcs = (mem, mbox, mbox, nfeat, aux)
    src_slices = (None, pl.ds(0, D), pl.ds(D, D), None, None)

    def start_gathers(c, buf, sem):
        idx = idx_all.at[pl.ds(c * _CH, _CH)]
        for j in range(NT):
            tbl, sl = srcs[j], src_slices[j]
            src = tbl.at[idx] if sl is None else tbl.at[idx, sl]
            pltpu.async_copy(src, buf.at[:, pl.ds(j * D, D)], sem)

    def wait_gathers(buf, sem):
        for j in range(NT):
            pltpu.make_async_copy(
                mem.at[idx_all.at[pl.ds(0, _CH)]],
                buf.at[:, pl.ds(j * D, D)], sem).wait()

    def start_outs(c, buf, sem):
        off = base + c * _CH
        pltpu.async_copy(buf, o_big.at[pl.ds(off, _CH)], sem)

    def drain_outs(buf, sem):
        pltpu.make_async_copy(buf, o_big.at[pl.ds(base, _CH)], sem).wait()

    start_gathers(0, b0, gsem0)

    def pair(g, carry):
        c0 = 2 * g
        wait_gathers(b0, gsem0)
        start_outs(c0, b0, osem0)

        @pl.when(g > 0)
        def _():
            drain_outs(b1, osem1)
        start_gathers(c0 + 1, b1, gsem1)
        wait_gathers(b1, gsem1)
        start_outs(c0 + 1, b1, osem1)
        drain_outs(b0, osem0)

        @pl.when(g < _NCHUNK // 2 - 1)
        def _():
            start_gathers(c0 + 2, b0, gsem0)
        return carry

    lax.fori_loop(0, _NCHUNK // 2, pair, 0)
    drain_outs(b1, osem1)


def _gather(nodes, mem, mailbox, nfeat, aux):
    f32 = jnp.float32
    run = pl.kernel(
        _sc_gather_body,
        mesh=plsc.VectorSubcoreMesh(core_axis_name="c", subcore_axis_name="s",
                                    num_cores=_NC),
        out_type=jax.ShapeDtypeStruct((E, NT * D), f32),
        scratch_types=[
            pltpu.VMEM((_PER_W,), jnp.int32),
            pltpu.VMEM((_CH, NT * D), f32),
            pltpu.VMEM((_CH, NT * D), f32),
            pltpu.SemaphoreType.DMA,
            pltpu.SemaphoreType.DMA,
            pltpu.SemaphoreType.DMA,
            pltpu.SemaphoreType.DMA,
        ],
    )
    return run(nodes, mem, mailbox, nfeat, aux)


def _gru_body(mlo_ref, mhi_ref, aux_ref, mem_ref, nfeat_ref,
              wt_ref, bt_ref, w1_ref, w2_ref, waux_ref, wit_ref,
              bih_ref, whh_ref, bhh_ref, h_ref):
    f32 = jnp.float32
    bf = jnp.bfloat16
    delta = aux_ref[:, DE:DE + 1]
    te = jnp.cos(delta * wt_ref[...] + bt_ref[...])
    gx = (jnp.dot(mlo_ref[...].astype(bf), w1_ref[...], preferred_element_type=f32)
          + jnp.dot(mhi_ref[...].astype(bf), w2_ref[...], preferred_element_type=f32)
          + jnp.dot(aux_ref[...].astype(bf), waux_ref[...], preferred_element_type=f32)
          + jnp.dot(te.astype(bf), wit_ref[...], preferred_element_type=f32)
          + bih_ref[...])
    h_prev = mem_ref[...]
    gh = (jnp.dot(h_prev.astype(bf), whh_ref[...], preferred_element_type=f32)
          + bhh_ref[...])
    r = jax.nn.sigmoid(gx[:, :D] + gh[:, :D])
    z = jax.nn.sigmoid(gx[:, D:2 * D] + gh[:, D:2 * D])
    n = jnp.tanh(gx[:, 2 * D:] + r * gh[:, 2 * D:])
    new_mem = (1.0 - z) * n + z * h_prev
    h_ref[...] = nfeat_ref[...] + new_mem


def _gru(big_g, w_t, b_t, W_ih, b_ih, W_hh, b_hh):
    R = 512
    grid = (E // R,)
    bf = jnp.bfloat16
    W_ihT = W_ih.T
    W_aux = jnp.zeros((D, 3 * D), jnp.float32).at[:DE].set(W_ihT[2 * D:2 * D + DE])
    col = lambda j: (lambda i: (i, j))
    full = lambda i: (0, 0)
    return pl.pallas_call(
        _gru_body,
        grid=grid,
        in_specs=[
            pl.BlockSpec((R, D), col(1)),
            pl.BlockSpec((R, D), col(2)),
            pl.BlockSpec((R, D), col(4)),
            pl.BlockSpec((R, D), col(0)),
            pl.BlockSpec((R, D), col(3)),
            pl.BlockSpec((1, DT), full),
            pl.BlockSpec((1, DT), full),
            pl.BlockSpec((D, 3 * D), full),
            pl.BlockSpec((D, 3 * D), full),
            pl.BlockSpec((D, 3 * D), full),
            pl.BlockSpec((DT, 3 * D), full),
            pl.BlockSpec((1, 3 * D), full),
            pl.BlockSpec((D, 3 * D), full),
            pl.BlockSpec((1, 3 * D), full),
        ],
        out_specs=pl.BlockSpec((R, D), lambda i: (i, 0)),
        out_shape=jax.ShapeDtypeStruct((E, D), jnp.float32),
    )(big_g, big_g, big_g, big_g, big_g,
      w_t.reshape(1, DT), b_t.reshape(1, DT),
      W_ihT[:D].astype(bf), W_ihT[D:2 * D].astype(bf), W_aux.astype(bf),
      W_ihT[2 * D + DE:].astype(bf),
      b_ih.reshape(1, 3 * D), W_hh.T.astype(bf), b_hh.reshape(1, 3 * D))


_RA = 256


def _attn_body(dsth_ref, srch_ref, dstt_ref, nbrt_ref, ef_ref,
               wt_ref, bt_ref, wqh_ref, wqt_ref,
               wkh_ref, wke_ref, wkt_ref, wvh_ref, wve_ref, wvt_ref,
               woh_ref, wo1_ref, wo2_ref, bo_ref, emb_ref):
    R = _RA
    f32 = jnp.float32
    bf = jnp.bfloat16
    wt = wt_ref[...]
    bt = bt_ref[...]
    dsth = dsth_ref[...]
    tz = jnp.cos(bt)
    q = (jnp.dot(dsth.astype(bf), wqh_ref[...], preferred_element_type=f32)
         + jnp.dot(tz.astype(bf), wqt_ref[...], preferred_element_type=f32))
    srch = srch_ref[...]
    ef = ef_ref[...]
    nbrt = nbrt_ref[...]
    dstt = dstt_ref[...]
    a1, a2, vs = [], [], []
    for k in range(K):
        sl = slice(k * R, (k + 1) * R)
        te_k = jnp.cos((dstt - nbrt[sl]) * wt + bt)
        s_k = srch[sl].astype(bf)
        e_k = ef[sl].astype(bf)
        t_k = te_k.astype(bf)
        kk = (jnp.dot(s_k, wkh_ref[...], preferred_element_type=f32)
              + jnp.dot(e_k, wke_ref[...], preferred_element_type=f32)
              + jnp.dot(t_k, wkt_ref[...], preferred_element_type=f32))
        vv = (jnp.dot(s_k, wvh_ref[...], preferred_element_type=f32)
              + jnp.dot(e_k, wve_ref[...], preferred_element_type=f32)
              + jnp.dot(t_k, wvt_ref[...], preferred_element_type=f32))
        p = q * kk
        a1.append(jnp.sum(p[:, :DH], axis=1, keepdims=True))
        a2.append(jnp.sum(p[:, DH:], axis=1, keepdims=True))
        vs.append(vv)
    scale = 1.0 / (DH ** 0.5)
    A1 = jnp.concatenate(a1, axis=1) * scale
    A2 = jnp.concatenate(a2, axis=1) * scale
    A1 = jnp.exp(A1 - jnp.max(A1, axis=1, keepdims=True))
    A2 = jnp.exp(A2 - jnp.max(A2, axis=1, keepdims=True))
    A1 = A1 / jnp.sum(A1, axis=1, keepdims=True)
    A2 = A2 / jnp.sum(A2, axis=1, keepdims=True)
    o1 = jnp.zeros((R, DH), f32)
    o2 = jnp.zeros((R, DH), f32)
    for k in range(K):
        o1 = o1 + A1[:, k:k + 1] * vs[k][:, :DH]
        o2 = o2 + A2[:, k:k + 1] * vs[k][:, DH:]
    emb = (jnp.dot(dsth.astype(bf), woh_ref[...], preferred_element_type=f32)
           + jnp.dot(o1.astype(bf), wo1_ref[...], preferred_element_type=f32)
           + jnp.dot(o2.astype(bf), wo2_ref[...], preferred_element_type=f32)
           + bo_ref[...])
    emb_ref[...] = jnp.maximum(emb, 0.0)


def _attn(h, dst_times, nbrt_km, ef_km, w_t, b_t, Wq, Wk, Wv, Wo, bo):
    grid = (B // _RA,)
    bf = jnp.bfloat16
    full = lambda i: (0, 0)
    nb = B // _RA
    return pl.pallas_call(
        _attn_body,
        grid=grid,
        in_specs=[
            pl.BlockSpec((_RA, D), lambda i: (i, 0)),
            pl.BlockSpec((_RA * K, D), lambda i: (i + 1, 0)),
            pl.BlockSpec((_RA, 1), lambda i: (i, 0)),
            pl.BlockSpec((_RA * K, 1), lambda i: (i, 0)),
            pl.BlockSpec((_RA * K, DE), lambda i: (i, 0)),
            pl.BlockSpec((1, DT), full),
            pl.BlockSpec((1, DT), full),
            pl.BlockSpec((D, D), full),
            pl.BlockSpec((DT, D), full),
            pl.BlockSpec((D, D), full),
            pl.BlockSpec((DE, D), full),
            pl.BlockSpec((DT, D), full),
            pl.BlockSpec((D, D), full),
            pl.BlockSpec((DE, D), full),
            pl.BlockSpec((DT, D), full),
            pl.BlockSpec((D, D), full),
            pl.BlockSpec((DH, D), full),
            pl.BlockSpec((DH, D), full),
            pl.BlockSpec((1, D), full),
        ],
        out_specs=pl.BlockSpec((_RA, D), lambda i: (i, 0)),
        out_shape=jax.ShapeDtypeStruct((B, D), jnp.float32),
    )(h, h, dst_times.reshape(B, 1), nbrt_km, ef_km,
      w_t.reshape(1, DT), b_t.reshape(1, DT),
      Wq[:D].astype(bf), Wq[D:].astype(bf),
      Wk[:D].astype(bf), Wk[D:D + DE].astype(bf), Wk[D + DE:].astype(bf),
      Wv[:D].astype(bf), Wv[D:D + DE].astype(bf), Wv[D + DE:].astype(bf),
      Wo[:D].astype(bf), Wo[D:D + DH].astype(bf), Wo[D + DH:].astype(bf),
      bo.reshape(1, D))


def _pred_body(src_ref, dst_ref, ws_ref, bs_ref, wd_ref, bd_ref, wo_ref, bo_ref,
               out_ref):
    f32 = jnp.float32
    hidden = (jnp.dot(src_ref[...], ws_ref[...], preferred_element_type=f32)
              + jnp.dot(dst_ref[...], wd_ref[...], preferred_element_type=f32)
              + bs_ref[...] + bd_ref[...])
    hidden = jnp.maximum(hidden, 0.0)
    out_ref[...] = jnp.dot(hidden, wo_ref[...], preferred_element_type=f32) + bo_ref[...]


def _pred(embed, W_src, b_src, W_dst, b_dst, W_out, b_out):
    Bh = B // 2
    return pl.pallas_call(
        _pred_body,
        grid=(1,),
        in_specs=[
            pl.BlockSpec((Bh, D), lambda i: (0, 0)),
            pl.BlockSpec((Bh, D), lambda i: (1, 0)),
            pl.BlockSpec((D, D), lambda i: (0, 0)),
            pl.BlockSpec((1, D), lambda i: (0, 0)),
            pl.BlockSpec((D, D), lambda i: (0, 0)),
            pl.BlockSpec((1, D), lambda i: (0, 0)),
            pl.BlockSpec((D, 1), lambda i: (0, 0)),
            pl.BlockSpec((1, 1), lambda i: (0, 0)),
        ],
        out_specs=pl.BlockSpec((Bh, 1), lambda i: (0, 0)),
        out_shape=jax.ShapeDtypeStruct((Bh, 1), jnp.float32),
    )(embed, embed, W_src, b_src.reshape(1, D), W_dst, b_dst.reshape(1, D),
      W_out, b_out.reshape(1, 1))


def kernel(dst_ids, src_ids, dst_times, nbr_times, efeat, mem, mem_time,
           mailbox, mail_time, nfeat, w_t, b_t, W_ih, b_ih, W_hh, b_hh,
           Wq, Wk, Wv, Wo, bo, W_src, b_src, W_dst, b_dst, W_out, b_out):
    nb = B // _RA
    src_km = src_ids.reshape(nb, _RA, K).transpose(0, 2, 1).reshape(-1)
    nbrt_km = nbr_times.reshape(nb, _RA, K).transpose(0, 2, 1).reshape(B * K, 1)
    ef_km = efeat.reshape(nb, _RA, K, DE).transpose(0, 2, 1, 3).reshape(B * K, DE)
    nodes = jnp.concatenate([dst_ids, src_km], axis=0).astype(jnp.int32)
    delta = mail_time - mem_time
    aux = jnp.concatenate(
        [mailbox[:, 2 * D:], delta[:, None],
         jnp.zeros((N, D - DE - 1), jnp.float32)], axis=1)
    big_g = _gather(nodes, mem, mailbox, nfeat, aux)
    h = _gru(big_g, w_t, b_t, W_ih, b_ih, W_hh, b_hh)
    embed = _attn(h, dst_times, nbrt_km, ef_km, w_t, b_t, Wq, Wk, Wv, Wo, bo)
    return _pred(embed, W_src, b_src, W_dst, b_dst, W_out, b_out)
